# baseline (speedup 1.0000x reference)
import jax
import jax.numpy as jnp
from jax.experimental import pallas as pl


def _mlp_body(z_ref, w1_ref, b1_ref, w2_ref, b2_ref, o_ref):
    z = z_ref[...]
    o = jnp.maximum(z @ w1_ref[...] + b1_ref[...], 0.0)
    out = o @ w2_ref[...] + b2_ref[...]
    o_ref[...] = out


def _gcn_conv(h, src2, dst2, norm, W, b):
    hw = h @ W
    msg = hw[src2] * norm[:, None]
    out = jax.ops.segment_sum(msg, dst2, num_segments=h.shape[0])
    return out + b


def kernel(x, edge_index, edge_attr, batch, Wn, bn, We, be, Wc, bc, Wo1, bo1, Wo2, bo2):
    n = x.shape[0]
    G = 256
    L = Wc.shape[0]
    src = edge_index[0]
    dst = edge_index[1]
    loop = jnp.arange(n, dtype=src.dtype)
    src2 = jnp.concatenate([src, loop])
    dst2 = jnp.concatenate([dst, loop])
    ones = jnp.ones((src2.shape[0],), dtype=x.dtype)
    deg = jax.ops.segment_sum(ones, dst2, num_segments=n)
    dinv = jnp.where(deg > 0, 1.0 / jnp.sqrt(deg), 0.0)
    norm = dinv[src2] * dinv[dst2]

    h = x @ Wn + bn
    for i in range(L):
        h = _gcn_conv(h, src2, dst2, norm, Wc[i], bc[i])
        h = jax.nn.relu(h)

    counts = jax.ops.segment_sum(jnp.ones((n,), dtype=h.dtype), batch, num_segments=G)
    x_mean = jax.ops.segment_sum(h, batch, num_segments=G) / jnp.maximum(counts, 1.0)[:, None]
    x_max = jax.ops.segment_max(h, batch, num_segments=G)
    x_max = jnp.where(counts[:, None] > 0, x_max, 0.0)
    z = jnp.concatenate([x_mean, x_max], axis=1)

    out2 = pl.pallas_call(
        _mlp_body,
        out_shape=jax.ShapeDtypeStruct((G, 1), jnp.float32),
    )(z, Wo1, bo1, Wo2, bo2)
    return out2[:, 0]


# speedup vs baseline: 9.0299x; 9.0299x over previous
import functools

import jax
import jax.numpy as jnp
from jax import lax
from jax.experimental import pallas as pl
from jax.experimental.pallas import tpu as pltpu
from jax.experimental.pallas import tpu_sc as plsc

N = 50000
NP = 51200
NT = NP + 128
E = 800000
EP = 819200
G = 256
GP = 272
H = 64
HH = 32
ROWS_PER_TILE = NT // 16
NODE_ROWS_PER_TILE = NP // 16

_mesh = plsc.VectorSubcoreMesh(core_axis_name="c", subcore_axis_name="s")
_sc_params = pltpu.CompilerParams(use_tc_tiling_on_sc=False)


@functools.partial(
    pl.kernel,
    out_type=jax.ShapeDtypeStruct((2, NT, 16), jnp.float32),
    mesh=_mesh,
    compiler_params=_sc_params,
    scratch_types=[
        pltpu.VMEM((200, 128), jnp.int32),
        pltpu.VMEM((128, 16), jnp.float32),
        pltpu.VMEM_SHARED((NT, 16), jnp.float32),
        pltpu.SemaphoreType.DMA,
    ],
)
def _sc_degrees(dst_hbm, zeros_hbm, ones_hbm, out_hbm, idx_d, ones_v, acc, sem):
    c = lax.axis_index("c")
    s = lax.axis_index("s")
    r0 = s * ROWS_PER_TILE
    pltpu.sync_copy(zeros_hbm, acc.at[pl.ds(r0, ROWS_PER_TILE)])
    pltpu.sync_copy(ones_hbm, ones_v)
    pltpu.sync_copy(dst_hbm.at[c].at[s], idx_d)
    plsc.subcore_barrier()

    def step(i, _):
        for k in range(8):
            j = i * 8 + k
            pltpu.sync_copy(ones_v, acc.at[idx_d.at[j]], add=True)
        return 0

    lax.fori_loop(0, 25, step, 0)
    plsc.subcore_barrier()
    pltpu.sync_copy(acc.at[pl.ds(r0, ROWS_PER_TILE)],
                    out_hbm.at[c].at[pl.ds(r0, ROWS_PER_TILE)])


@functools.partial(
    pl.kernel,
    out_type=jax.ShapeDtypeStruct((4, NT, 16), jnp.float32),
    mesh=_mesh,
    compiler_params=_sc_params,
    scratch_types=[
        pltpu.VMEM((200, 128), jnp.int32),
        pltpu.VMEM((200, 128), jnp.int32),
        pltpu.VMEM((128, 16), jnp.float32),
        pltpu.VMEM_SHARED((NT, 16), jnp.float32),
        pltpu.SemaphoreType.DMA,
    ],
)
def _sc_edge_scatter(g_hbm, src_hbm, dst_hbm, zeros_hbm, out_hbm,
                     idx_s, idx_d, rows, acc, sem):
    c = lax.axis_index("c")
    s = lax.axis_index("s")
    r0 = s * ROWS_PER_TILE
    for p in range(2):
        q = c * 2 + p
        pltpu.sync_copy(zeros_hbm, acc.at[pl.ds(r0, ROWS_PER_TILE)])
        plsc.subcore_barrier()
        gq = g_hbm.at[q]

        def step(i, _):
            for k in range(8):
                j = i * 8 + k
                pltpu.async_copy(gq.at[idx_s.at[j]], rows, sem).wait()
                pltpu.sync_copy(rows, acc.at[idx_d.at[j]], add=True)
            return 0

        for half in range(2):
            pltpu.sync_copy(src_hbm.at[s].at[pl.ds(half * 200, 200)], idx_s)
            pltpu.sync_copy(dst_hbm.at[s].at[pl.ds(half * 200, 200)], idx_d)
            lax.fori_loop(0, 25, step, 0)
        plsc.subcore_barrier()
        pltpu.sync_copy(acc.at[pl.ds(r0, ROWS_PER_TILE)],
                        out_hbm.at[q].at[pl.ds(r0, ROWS_PER_TILE)])
        plsc.subcore_barrier()


@functools.partial(
    pl.kernel,
    out_type=[
        jax.ShapeDtypeStruct((2, 16, GP, HH), jnp.float32),
        jax.ShapeDtypeStruct((2, 16, GP, HH), jnp.float32),
        jax.ShapeDtypeStruct((2, 16, GP, 16), jnp.float32),
    ],
    mesh=_mesh,
    compiler_params=_sc_params,
    scratch_types=[
        pltpu.VMEM((25, 128), jnp.int32),
        pltpu.VMEM((128, HH), jnp.float32),
        pltpu.VMEM((GP, HH), jnp.float32),
        pltpu.VMEM((GP, HH), jnp.float32),
        pltpu.VMEM((GP, 16), jnp.float32),
        pltpu.SemaphoreType.DMA,
    ],
)
def _sc_pool(h_hbm, batch_hbm, sum_hbm, max_hbm, cnt_hbm,
             bidx, win, sacc, macc, cacc, sem):
    c = lax.axis_index("c")
    s = lax.axis_index("s")
    pltpu.sync_copy(batch_hbm.at[s], bidx)
    zero16 = jnp.zeros((16,), jnp.float32)
    neg16 = jnp.full((16,), -1e30, jnp.float32)

    def init(i, _):
        sacc[i, pl.ds(0, 16)] = zero16
        sacc[i, pl.ds(16, 16)] = zero16
        macc[i, pl.ds(0, 16)] = neg16
        macc[i, pl.ds(16, 16)] = neg16
        cacc[i, pl.ds(0, 16)] = zero16
        return 0

    lax.fori_loop(0, GP, init, 0)
    hc = h_hbm.at[c]
    base = s * NODE_ROWS_PER_TILE

    def window(w, _):
        pltpu.sync_copy(hc.at[pl.ds(base + w * 128, 128)], win)

        def rowgrp(rr, _):
            bvec = bidx[w, pl.ds(rr * 16, 16)]
            for k in range(16):
                b = bvec[k]
                r = rr * 16 + k
                v0 = win[r, pl.ds(0, 16)]
                v1 = win[r, pl.ds(16, 16)]
                sacc[b, pl.ds(0, 16)] = sacc[b, pl.ds(0, 16)] + v0
                sacc[b, pl.ds(16, 16)] = sacc[b, pl.ds(16, 16)] + v1
                macc[b, pl.ds(0, 16)] = jnp.maximum(macc[b, pl.ds(0, 16)], v0)
                macc[b, pl.ds(16, 16)] = jnp.maximum(macc[b, pl.ds(16, 16)], v1)
                cacc[b, pl.ds(0, 16)] = cacc[b, pl.ds(0, 16)] + 1.0
            return 0

        lax.fori_loop(0, 8, rowgrp, 0)
        return 0

    lax.fori_loop(0, 25, window, 0)
    pltpu.sync_copy(sacc, sum_hbm.at[c].at[s])
    pltpu.sync_copy(macc, max_hbm.at[c].at[s])
    pltpu.sync_copy(cacc, cnt_hbm.at[c].at[s])


_BR = 1024
_GRID = NP // _BR


def _tc_embed_body(x_ref, cnt_ref, wn_ref, bn_ref, w0_ref, g_ref, dinv_ref):
    cnt = cnt_ref[...]
    deg = 1.0 + cnt[0, :, 0] + cnt[1, :, 0]
    dinv = lax.rsqrt(deg)
    h = x_ref[...] @ wn_ref[...] + bn_ref[...]
    g = (h @ w0_ref[...]) * dinv[:, None]
    for q in range(4):
        g_ref[q] = g[:, q * 16:(q + 1) * 16]
    dinv_ref[...] = dinv


def _tc_embed(x_p, cnt, Wn, bn, W0):
    return pl.pallas_call(
        _tc_embed_body,
        grid=(_GRID,),
        in_specs=[
            pl.BlockSpec((_BR, 128), lambda i: (i, 0)),
            pl.BlockSpec((2, _BR, 16), lambda i: (0, i, 0)),
            pl.BlockSpec((128, H), lambda i: (0, 0)),
            pl.BlockSpec((H,), lambda i: (0,)),
            pl.BlockSpec((H, H), lambda i: (0, 0)),
        ],
        out_specs=[
            pl.BlockSpec((4, _BR, 16), lambda i: (0, i, 0)),
            pl.BlockSpec((_BR,), lambda i: (i,)),
        ],
        out_shape=[
            jax.ShapeDtypeStruct((4, NP, 16), jnp.float32),
            jax.ShapeDtypeStruct((NP,), jnp.float32),
        ],
    )(x_p, cnt, Wn, bn, W0)


def _tc_mid_body(s_ref, g_ref, dinv_ref, b_ref, w_ref, o_ref):
    sv = jnp.concatenate([s_ref[q] for q in range(4)], axis=1)
    gv = jnp.concatenate([g_ref[q] for q in range(4)], axis=1)
    dinv = dinv_ref[...]
    hn = jnp.maximum((sv + gv) * dinv[:, None] + b_ref[...], 0.0)
    gn = (hn @ w_ref[...]) * dinv[:, None]
    for q in range(4):
        o_ref[q] = gn[:, q * 16:(q + 1) * 16]


def _tc_mid(s_quad, g_quad, dinv, b, Wnext):
    return pl.pallas_call(
        _tc_mid_body,
        grid=(_GRID,),
        in_specs=[
            pl.BlockSpec((4, _BR, 16), lambda i: (0, i, 0)),
            pl.BlockSpec((4, _BR, 16), lambda i: (0, i, 0)),
            pl.BlockSpec((_BR,), lambda i: (i,)),
            pl.BlockSpec((H,), lambda i: (0,)),
            pl.BlockSpec((H, H), lambda i: (0, 0)),
        ],
        out_specs=pl.BlockSpec((4, _BR, 16), lambda i: (0, i, 0)),
        out_shape=jax.ShapeDtypeStruct((4, NP, 16), jnp.float32),
    )(s_quad, g_quad, dinv, b, Wnext)


def _tc_last_body(s_ref, g_ref, dinv_ref, b_ref, o_ref):
    sv = jnp.concatenate([s_ref[q] for q in range(4)], axis=1)
    gv = jnp.concatenate([g_ref[q] for q in range(4)], axis=1)
    dinv = dinv_ref[...]
    hn = jnp.maximum((sv + gv) * dinv[:, None] + b_ref[...], 0.0)
    o_ref[0] = hn[:, :HH]
    o_ref[1] = hn[:, HH:]


def _tc_last(s_quad, g_quad, dinv, b):
    return pl.pallas_call(
        _tc_last_body,
        grid=(_GRID,),
        in_specs=[
            pl.BlockSpec((4, _BR, 16), lambda i: (0, i, 0)),
            pl.BlockSpec((4, _BR, 16), lambda i: (0, i, 0)),
            pl.BlockSpec((_BR,), lambda i: (i,)),
            pl.BlockSpec((H,), lambda i: (0,)),
        ],
        out_specs=pl.BlockSpec((2, _BR, HH), lambda i: (0, i, 0)),
        out_shape=jax.ShapeDtypeStruct((2, NP, HH), jnp.float32),
    )(s_quad, g_quad, dinv, b)


def _tc_head_body(sum_ref, max_ref, cnt_ref, w1_ref, b1_ref, w2_ref, b2_ref,
                  o_ref):
    sm = sum_ref[...]
    mx = max_ref[...]
    ct = cnt_ref[...]
    counts = jnp.sum(ct[0], axis=0)[:G, 0]
    s0 = jnp.sum(sm[0], axis=0)[:G]
    s1 = jnp.sum(sm[1], axis=0)[:G]
    m0 = jnp.max(mx[0], axis=0)[:G]
    m1 = jnp.max(mx[1], axis=0)[:G]
    inv = 1.0 / jnp.maximum(counts, 1.0)
    nz = counts > 0.0
    m0 = jnp.where(nz[:, None], m0, 0.0)
    m1 = jnp.where(nz[:, None], m1, 0.0)
    z = jnp.concatenate([s0 * inv[:, None], s1 * inv[:, None], m0, m1], axis=1)
    o = jnp.maximum(z @ w1_ref[...] + b1_ref[...], 0.0)
    o_ref[...] = o @ w2_ref[...] + b2_ref[...]


def _tc_head(sums, maxs, cnts, Wo1, bo1, Wo2, bo2):
    return pl.pallas_call(
        _tc_head_body,
        out_shape=jax.ShapeDtypeStruct((G, 1), jnp.float32),
    )(sums, maxs, cnts, Wo1, bo1, Wo2, bo2)


def kernel(x, edge_index, edge_attr, batch, Wn, bn, We, be, Wc, bc, Wo1, bo1,
           Wo2, bo2):
    L = Wc.shape[0]
    src = edge_index[0]
    dst = edge_index[1]
    pad = jnp.arange(EP - E, dtype=jnp.int32)
    src_p = jnp.concatenate([src, pad % 16]).reshape(16, 400, 128)
    dst_p = jnp.concatenate([dst, NP + (pad % 32)])
    dst_sc = dst_p.reshape(16, 400, 128)
    dst_deg = dst_p.reshape(2, 16, 200, 128)
    x_p = jnp.zeros((NP, 128), jnp.float32).at[:N].set(x)
    bpad = G + jnp.arange(NP - N, dtype=jnp.int32) % 16
    batch_p = jnp.concatenate([batch, bpad]).reshape(16, 25, 128)

    zeros16 = jnp.zeros((ROWS_PER_TILE, 16), jnp.float32)
    ones128 = jnp.ones((128, 16), jnp.float32)

    cnt = _sc_degrees(dst_deg, zeros16, ones128)
    g_quad, dinv = _tc_embed(x_p, cnt, Wn, bn, Wc[0])
    for i in range(L):
        s_quad = _sc_edge_scatter(g_quad, src_p, dst_sc, zeros16)
        if i < L - 1:
            g_quad = _tc_mid(s_quad, g_quad, dinv, bc[i], Wc[i + 1])
        else:
            h_pair = _tc_last(s_quad, g_quad, dinv, bc[i])
    sums, maxs, cnts = _sc_pool(h_pair, batch_p)
    out = _tc_head(sums, maxs, cnts, Wo1, bo1, Wo2, bo2)
    return out[:, 0]


# speedup vs baseline: 15.8260x; 1.7526x over previous
import functools

import jax
import jax.numpy as jnp
from jax import lax
from jax.experimental import pallas as pl
from jax.experimental.pallas import tpu as pltpu
from jax.experimental.pallas import tpu_sc as plsc

N = 50000
NP = 51200
NT = NP + 128
E = 800000
EP = 819200
G = 256
GP = 272
H = 64
HH = 32
ROWS_PER_TILE = NT // 16
NODE_ROWS_PER_TILE = NP // 16

_mesh = plsc.VectorSubcoreMesh(core_axis_name="c", subcore_axis_name="s")
_sc_params = pltpu.CompilerParams(use_tc_tiling_on_sc=False)


@functools.partial(
    pl.kernel,
    out_type=jax.ShapeDtypeStruct((2, NT, 16), jnp.float32),
    mesh=_mesh,
    compiler_params=_sc_params,
    scratch_types=[
        pltpu.VMEM((200, 128), jnp.int32),
        pltpu.VMEM((128, 16), jnp.float32),
        pltpu.VMEM_SHARED((NT, 16), jnp.float32),
        pltpu.SemaphoreType.DMA,
    ],
)
def _sc_degrees(dst_hbm, zeros_hbm, ones_hbm, out_hbm, idx_d, ones_v, acc, sem):
    c = lax.axis_index("c")
    s = lax.axis_index("s")
    r0 = s * ROWS_PER_TILE
    pltpu.sync_copy(zeros_hbm, acc.at[pl.ds(r0, ROWS_PER_TILE)])
    pltpu.sync_copy(ones_hbm, ones_v)
    pltpu.sync_copy(dst_hbm.at[c].at[s], idx_d)
    plsc.subcore_barrier()

    def step(i, _):
        for k in range(8):
            j = i * 8 + k
            pltpu.sync_copy(ones_v, acc.at[idx_d.at[j]], add=True)
        return 0

    lax.fori_loop(0, 25, step, 0)
    plsc.subcore_barrier()
    pltpu.sync_copy(acc.at[pl.ds(r0, ROWS_PER_TILE)],
                    out_hbm.at[c].at[pl.ds(r0, ROWS_PER_TILE)])


@functools.partial(
    pl.kernel,
    out_type=jax.ShapeDtypeStruct((4, NT, 16), jnp.float32),
    mesh=_mesh,
    compiler_params=_sc_params,
    scratch_types=[
        pltpu.VMEM((200, 128), jnp.int32),
        pltpu.VMEM((200, 128), jnp.int32),
        pltpu.VMEM((8, 128, 16), jnp.float32),
        pltpu.VMEM_SHARED((NT, 16), jnp.float32),
        pltpu.SemaphoreType.DMA,
    ],
)
def _sc_edge_scatter(g_hbm, src_hbm, dst_hbm, zeros_hbm, out_hbm,
                     idx_s, idx_d, rows, acc, sem):
    c = lax.axis_index("c")
    s = lax.axis_index("s")
    r0 = s * ROWS_PER_TILE
    for p in range(2):
        q = c * 2 + p
        pltpu.sync_copy(zeros_hbm, acc.at[pl.ds(r0, ROWS_PER_TILE)])
        plsc.subcore_barrier()
        gq = g_hbm.at[q]
        dummy = gq.at[pl.ds(0, 128)]

        def step(i, _):
            for b in range(8):
                j = i * 8 + b
                pltpu.make_async_copy(dummy, rows.at[b], sem).wait()
                pltpu.sync_copy(rows.at[b], acc.at[idx_d.at[j]], add=True)
                pltpu.async_copy(gq.at[idx_s.at[j + 8]], rows.at[b], sem)
            return 0

        for half in range(2):
            pltpu.sync_copy(src_hbm.at[s].at[pl.ds(half * 200, 200)], idx_s)
            pltpu.sync_copy(dst_hbm.at[s].at[pl.ds(half * 200, 200)], idx_d)
            for b in range(8):
                pltpu.async_copy(gq.at[idx_s.at[b]], rows.at[b], sem)
            lax.fori_loop(0, 24, step, 0)
            for b in range(8):
                j = 192 + b
                pltpu.make_async_copy(dummy, rows.at[b], sem).wait()
                pltpu.sync_copy(rows.at[b], acc.at[idx_d.at[j]], add=True)
        plsc.subcore_barrier()
        pltpu.sync_copy(acc.at[pl.ds(r0, ROWS_PER_TILE)],
                        out_hbm.at[q].at[pl.ds(r0, ROWS_PER_TILE)])
        plsc.subcore_barrier()


@functools.partial(
    pl.kernel,
    out_type=[
        jax.ShapeDtypeStruct((2, 16, GP, HH), jnp.float32),
        jax.ShapeDtypeStruct((2, 16, GP, HH), jnp.float32),
        jax.ShapeDtypeStruct((2, 16, GP, 16), jnp.float32),
    ],
    mesh=_mesh,
    compiler_params=_sc_params,
    scratch_types=[
        pltpu.VMEM((25, 128), jnp.int32),
        pltpu.VMEM((128, HH), jnp.float32),
        pltpu.VMEM((GP, HH), jnp.float32),
        pltpu.VMEM((GP, HH), jnp.float32),
        pltpu.VMEM((GP, 16), jnp.float32),
        pltpu.SemaphoreType.DMA,
    ],
)
def _sc_pool(h_hbm, batch_hbm, sum_hbm, max_hbm, cnt_hbm,
             bidx, win, sacc, macc, cacc, sem):
    c = lax.axis_index("c")
    s = lax.axis_index("s")
    pltpu.sync_copy(batch_hbm.at[s], bidx)
    zero16 = jnp.zeros((16,), jnp.float32)
    neg16 = jnp.full((16,), -1e30, jnp.float32)

    def init(i, _):
        sacc[i, pl.ds(0, 16)] = zero16
        sacc[i, pl.ds(16, 16)] = zero16
        macc[i, pl.ds(0, 16)] = neg16
        macc[i, pl.ds(16, 16)] = neg16
        cacc[i, pl.ds(0, 16)] = zero16
        return 0

    lax.fori_loop(0, GP, init, 0)
    hc = h_hbm.at[c]
    base = s * NODE_ROWS_PER_TILE

    def window(w, _):
        pltpu.sync_copy(hc.at[pl.ds(base + w * 128, 128)], win)

        def rowgrp(rr, _):
            bvec = bidx[w, pl.ds(rr * 16, 16)]
            for k in range(16):
                b = bvec[k]
                r = rr * 16 + k
                v0 = win[r, pl.ds(0, 16)]
                v1 = win[r, pl.ds(16, 16)]
                sacc[b, pl.ds(0, 16)] = sacc[b, pl.ds(0, 16)] + v0
                sacc[b, pl.ds(16, 16)] = sacc[b, pl.ds(16, 16)] + v1
                macc[b, pl.ds(0, 16)] = jnp.maximum(macc[b, pl.ds(0, 16)], v0)
                macc[b, pl.ds(16, 16)] = jnp.maximum(macc[b, pl.ds(16, 16)], v1)
                cacc[b, pl.ds(0, 16)] = cacc[b, pl.ds(0, 16)] + 1.0
            return 0

        lax.fori_loop(0, 8, rowgrp, 0)
        return 0

    lax.fori_loop(0, 25, window, 0)
    pltpu.sync_copy(sacc, sum_hbm.at[c].at[s])
    pltpu.sync_copy(macc, max_hbm.at[c].at[s])
    pltpu.sync_copy(cacc, cnt_hbm.at[c].at[s])


_BR = 1024
_GRID = NP // _BR


def _tc_embed_body(x_ref, cnt_ref, wn_ref, bn_ref, w0_ref, g_ref, dinv_ref):
    cnt = cnt_ref[...]
    deg = 1.0 + cnt[0, :, 0] + cnt[1, :, 0]
    dinv = lax.rsqrt(deg)
    h = x_ref[...] @ wn_ref[...] + bn_ref[...]
    g = (h @ w0_ref[...]) * dinv[:, None]
    for q in range(4):
        g_ref[q] = g[:, q * 16:(q + 1) * 16]
    dinv_ref[...] = dinv


def _tc_embed(x_p, cnt, Wn, bn, W0):
    return pl.pallas_call(
        _tc_embed_body,
        grid=(_GRID,),
        in_specs=[
            pl.BlockSpec((_BR, 128), lambda i: (i, 0)),
            pl.BlockSpec((2, _BR, 16), lambda i: (0, i, 0)),
            pl.BlockSpec((128, H), lambda i: (0, 0)),
            pl.BlockSpec((H,), lambda i: (0,)),
            pl.BlockSpec((H, H), lambda i: (0, 0)),
        ],
        out_specs=[
            pl.BlockSpec((4, _BR, 16), lambda i: (0, i, 0)),
            pl.BlockSpec((_BR,), lambda i: (i,)),
        ],
        out_shape=[
            jax.ShapeDtypeStruct((4, NP, 16), jnp.float32),
            jax.ShapeDtypeStruct((NP,), jnp.float32),
        ],
    )(x_p, cnt, Wn, bn, W0)


def _tc_mid_body(s_ref, g_ref, dinv_ref, b_ref, w_ref, o_ref):
    sv = jnp.concatenate([s_ref[q] for q in range(4)], axis=1)
    gv = jnp.concatenate([g_ref[q] for q in range(4)], axis=1)
    dinv = dinv_ref[...]
    hn = jnp.maximum((sv + gv) * dinv[:, None] + b_ref[...], 0.0)
    gn = (hn @ w_ref[...]) * dinv[:, None]
    for q in range(4):
        o_ref[q] = gn[:, q * 16:(q + 1) * 16]


def _tc_mid(s_quad, g_quad, dinv, b, Wnext):
    return pl.pallas_call(
        _tc_mid_body,
        grid=(_GRID,),
        in_specs=[
            pl.BlockSpec((4, _BR, 16), lambda i: (0, i, 0)),
            pl.BlockSpec((4, _BR, 16), lambda i: (0, i, 0)),
            pl.BlockSpec((_BR,), lambda i: (i,)),
            pl.BlockSpec((H,), lambda i: (0,)),
            pl.BlockSpec((H, H), lambda i: (0, 0)),
        ],
        out_specs=pl.BlockSpec((4, _BR, 16), lambda i: (0, i, 0)),
        out_shape=jax.ShapeDtypeStruct((4, NP, 16), jnp.float32),
    )(s_quad, g_quad, dinv, b, Wnext)


def _tc_last_body(s_ref, g_ref, dinv_ref, b_ref, o_ref):
    sv = jnp.concatenate([s_ref[q] for q in range(4)], axis=1)
    gv = jnp.concatenate([g_ref[q] for q in range(4)], axis=1)
    dinv = dinv_ref[...]
    hn = jnp.maximum((sv + gv) * dinv[:, None] + b_ref[...], 0.0)
    o_ref[0] = hn[:, :HH]
    o_ref[1] = hn[:, HH:]


def _tc_last(s_quad, g_quad, dinv, b):
    return pl.pallas_call(
        _tc_last_body,
        grid=(_GRID,),
        in_specs=[
            pl.BlockSpec((4, _BR, 16), lambda i: (0, i, 0)),
            pl.BlockSpec((4, _BR, 16), lambda i: (0, i, 0)),
            pl.BlockSpec((_BR,), lambda i: (i,)),
            pl.BlockSpec((H,), lambda i: (0,)),
        ],
        out_specs=pl.BlockSpec((2, _BR, HH), lambda i: (0, i, 0)),
        out_shape=jax.ShapeDtypeStruct((2, NP, HH), jnp.float32),
    )(s_quad, g_quad, dinv, b)


def _tc_head_body(sum_ref, max_ref, cnt_ref, w1_ref, b1_ref, w2_ref, b2_ref,
                  o_ref):
    sm = sum_ref[...]
    mx = max_ref[...]
    ct = cnt_ref[...]
    counts = jnp.sum(ct[0], axis=0)[:G, 0]
    s0 = jnp.sum(sm[0], axis=0)[:G]
    s1 = jnp.sum(sm[1], axis=0)[:G]
    m0 = jnp.max(mx[0], axis=0)[:G]
    m1 = jnp.max(mx[1], axis=0)[:G]
    inv = 1.0 / jnp.maximum(counts, 1.0)
    nz = counts > 0.0
    m0 = jnp.where(nz[:, None], m0, 0.0)
    m1 = jnp.where(nz[:, None], m1, 0.0)
    z = jnp.concatenate([s0 * inv[:, None], s1 * inv[:, None], m0, m1], axis=1)
    o = jnp.maximum(z @ w1_ref[...] + b1_ref[...], 0.0)
    o_ref[...] = o @ w2_ref[...] + b2_ref[...]


def _tc_head(sums, maxs, cnts, Wo1, bo1, Wo2, bo2):
    return pl.pallas_call(
        _tc_head_body,
        out_shape=jax.ShapeDtypeStruct((G, 1), jnp.float32),
    )(sums, maxs, cnts, Wo1, bo1, Wo2, bo2)


def kernel(x, edge_index, edge_attr, batch, Wn, bn, We, be, Wc, bc, Wo1, bo1,
           Wo2, bo2):
    L = Wc.shape[0]
    src = edge_index[0]
    dst = edge_index[1]
    pad = jnp.arange(EP - E, dtype=jnp.int32)
    src_p = jnp.concatenate([src, pad % 16]).reshape(16, 400, 128)
    dst_p = jnp.concatenate([dst, NP + (pad % 32)])
    dst_sc = dst_p.reshape(16, 400, 128)
    dst_deg = dst_p.reshape(2, 16, 200, 128)
    x_p = jnp.zeros((NP, 128), jnp.float32).at[:N].set(x)
    bpad = G + jnp.arange(NP - N, dtype=jnp.int32) % 16
    batch_p = jnp.concatenate([batch, bpad]).reshape(16, 25, 128)

    zeros16 = jnp.zeros((ROWS_PER_TILE, 16), jnp.float32)
    ones128 = jnp.ones((128, 16), jnp.float32)

    cnt = _sc_degrees(dst_deg, zeros16, ones128)
    g_quad, dinv = _tc_embed(x_p, cnt, Wn, bn, Wc[0])
    for i in range(L):
        s_quad = _sc_edge_scatter(g_quad, src_p, dst_sc, zeros16)
        if i < L - 1:
            g_quad = _tc_mid(s_quad, g_quad, dinv, bc[i], Wc[i + 1])
        else:
            h_pair = _tc_last(s_quad, g_quad, dinv, bc[i])
    sums, maxs, cnts = _sc_pool(h_pair, batch_p)
    out = _tc_head(sums, maxs, cnts, Wo1, bo1, Wo2, bo2)
    return out[:, 0]


# speedup vs baseline: 21.5536x; 1.3619x over previous
import functools

import jax
import jax.numpy as jnp
from jax import lax
from jax.experimental import pallas as pl
from jax.experimental.pallas import tpu as pltpu
from jax.experimental.pallas import tpu_sc as plsc

N = 50000
NP = 51200
NT = NP + 128
E = 800000
EP = 819200
G = 256
GP = 272
H = 64
HH = 32
ROWS_PER_TILE = NT // 16
NODE_ROWS_PER_TILE = NP // 16

_mesh = plsc.VectorSubcoreMesh(core_axis_name="c", subcore_axis_name="s")
_sc_params = pltpu.CompilerParams(use_tc_tiling_on_sc=False)


@functools.partial(
    pl.kernel,
    out_type=jax.ShapeDtypeStruct((2, NT, 16), jnp.float32),
    mesh=_mesh,
    compiler_params=_sc_params,
    scratch_types=[
        pltpu.VMEM((200, 128), jnp.int32),
        pltpu.VMEM((128, 16), jnp.float32),
        pltpu.VMEM_SHARED((NT, 16), jnp.float32),
        pltpu.SemaphoreType.DMA,
    ],
)
def _sc_degrees(dst_hbm, zeros_hbm, ones_hbm, out_hbm, idx_d, ones_v, acc, sem):
    c = lax.axis_index("c")
    s = lax.axis_index("s")
    r0 = s * ROWS_PER_TILE
    pltpu.sync_copy(zeros_hbm, acc.at[pl.ds(r0, ROWS_PER_TILE)])
    pltpu.sync_copy(ones_hbm, ones_v)
    pltpu.sync_copy(dst_hbm.at[c].at[s], idx_d)
    plsc.subcore_barrier()

    def step(i, _):
        for k in range(8):
            j = i * 8 + k
            pltpu.sync_copy(ones_v, acc.at[idx_d.at[j]], add=True)
        return 0

    lax.fori_loop(0, 25, step, 0)
    plsc.subcore_barrier()
    pltpu.sync_copy(acc.at[pl.ds(r0, ROWS_PER_TILE)],
                    out_hbm.at[c].at[pl.ds(r0, ROWS_PER_TILE)])


@functools.partial(
    pl.kernel,
    out_type=jax.ShapeDtypeStruct((2, NT, HH), jnp.float32),
    mesh=_mesh,
    compiler_params=_sc_params,
    scratch_types=[
        pltpu.VMEM((40, 128), jnp.int32),
        pltpu.VMEM((40, 128), jnp.int32),
        pltpu.VMEM((4, 128, HH), jnp.float32),
        pltpu.VMEM_SHARED((NT, HH), jnp.float32),
        pltpu.SemaphoreType.DMA,
    ],
)
def _sc_edge_scatter(g_hbm, src_hbm, dst_hbm, zeros_hbm, out_hbm,
                     idx_s, idx_d, rows, acc, sem):
    c = lax.axis_index("c")
    s = lax.axis_index("s")
    r0 = s * ROWS_PER_TILE
    pltpu.sync_copy(zeros_hbm, acc.at[pl.ds(r0, ROWS_PER_TILE)])
    plsc.subcore_barrier()
    gq = g_hbm.at[c]
    dummy = gq.at[pl.ds(0, 128)]

    def step(i, _):
        for b in range(4):
            j = i * 4 + b
            pltpu.make_async_copy(dummy, rows.at[b], sem).wait()
            pltpu.sync_copy(rows.at[b], acc.at[idx_d.at[j]], add=True)
            pltpu.async_copy(gq.at[idx_s.at[j + 4]], rows.at[b], sem)
        return 0

    for chunk in range(10):
        pltpu.sync_copy(src_hbm.at[s].at[pl.ds(chunk * 40, 40)], idx_s)
        pltpu.sync_copy(dst_hbm.at[s].at[pl.ds(chunk * 40, 40)], idx_d)
        for b in range(4):
            pltpu.async_copy(gq.at[idx_s.at[b]], rows.at[b], sem)
        lax.fori_loop(0, 9, step, 0)
        for b in range(4):
            j = 36 + b
            pltpu.make_async_copy(dummy, rows.at[b], sem).wait()
            pltpu.sync_copy(rows.at[b], acc.at[idx_d.at[j]], add=True)
    plsc.subcore_barrier()
    pltpu.sync_copy(acc.at[pl.ds(r0, ROWS_PER_TILE)],
                    out_hbm.at[c].at[pl.ds(r0, ROWS_PER_TILE)])


@functools.partial(
    pl.kernel,
    out_type=[
        jax.ShapeDtypeStruct((2, 16, GP, HH), jnp.float32),
        jax.ShapeDtypeStruct((2, 16, GP, HH), jnp.float32),
        jax.ShapeDtypeStruct((2, 16, GP, 16), jnp.float32),
    ],
    mesh=_mesh,
    compiler_params=_sc_params,
    scratch_types=[
        pltpu.VMEM((25, 128), jnp.int32),
        pltpu.VMEM((128, HH), jnp.float32),
        pltpu.VMEM((GP, HH), jnp.float32),
        pltpu.VMEM((GP, HH), jnp.float32),
        pltpu.VMEM((GP, 16), jnp.float32),
        pltpu.SemaphoreType.DMA,
    ],
)
def _sc_pool(h_hbm, batch_hbm, sum_hbm, max_hbm, cnt_hbm,
             bidx, win, sacc, macc, cacc, sem):
    c = lax.axis_index("c")
    s = lax.axis_index("s")
    pltpu.sync_copy(batch_hbm.at[s], bidx)
    zero16 = jnp.zeros((16,), jnp.float32)
    neg16 = jnp.full((16,), -1e30, jnp.float32)

    def init(i, _):
        sacc[i, pl.ds(0, 16)] = zero16
        sacc[i, pl.ds(16, 16)] = zero16
        macc[i, pl.ds(0, 16)] = neg16
        macc[i, pl.ds(16, 16)] = neg16
        cacc[i, pl.ds(0, 16)] = zero16
        return 0

    lax.fori_loop(0, GP, init, 0)
    hc = h_hbm.at[c]
    base = s * NODE_ROWS_PER_TILE

    def window(w, _):
        pltpu.sync_copy(hc.at[pl.ds(base + w * 128, 128)], win)

        def rowgrp(rr, _):
            bvec = bidx[w, pl.ds(rr * 16, 16)]
            for k in range(16):
                b = bvec[k]
                r = rr * 16 + k
                v0 = win[r, pl.ds(0, 16)]
                v1 = win[r, pl.ds(16, 16)]
                sacc[b, pl.ds(0, 16)] = sacc[b, pl.ds(0, 16)] + v0
                sacc[b, pl.ds(16, 16)] = sacc[b, pl.ds(16, 16)] + v1
                macc[b, pl.ds(0, 16)] = jnp.maximum(macc[b, pl.ds(0, 16)], v0)
                macc[b, pl.ds(16, 16)] = jnp.maximum(macc[b, pl.ds(16, 16)], v1)
                cacc[b, pl.ds(0, 16)] = cacc[b, pl.ds(0, 16)] + 1.0
            return 0

        lax.fori_loop(0, 8, rowgrp, 0)
        return 0

    lax.fori_loop(0, 25, window, 0)
    pltpu.sync_copy(sacc, sum_hbm.at[c].at[s])
    pltpu.sync_copy(macc, max_hbm.at[c].at[s])
    pltpu.sync_copy(cacc, cnt_hbm.at[c].at[s])


_BR = 1024
_GRID = NP // _BR


def _tc_embed_body(x_ref, cnt_ref, wn_ref, bn_ref, w0_ref, g_ref, dinv_ref):
    cnt = cnt_ref[...]
    deg = 1.0 + cnt[0, :, 0] + cnt[1, :, 0]
    dinv = lax.rsqrt(deg)
    h = x_ref[...] @ wn_ref[...] + bn_ref[...]
    g = (h @ w0_ref[...]) * dinv[:, None]
    for q in range(2):
        g_ref[q] = g[:, q * HH:(q + 1) * HH]
    dinv_ref[...] = dinv


def _tc_embed(x_p, cnt, Wn, bn, W0):
    return pl.pallas_call(
        _tc_embed_body,
        grid=(_GRID,),
        in_specs=[
            pl.BlockSpec((_BR, 128), lambda i: (i, 0)),
            pl.BlockSpec((2, _BR, 16), lambda i: (0, i, 0)),
            pl.BlockSpec((128, H), lambda i: (0, 0)),
            pl.BlockSpec((H,), lambda i: (0,)),
            pl.BlockSpec((H, H), lambda i: (0, 0)),
        ],
        out_specs=[
            pl.BlockSpec((2, _BR, HH), lambda i: (0, i, 0)),
            pl.BlockSpec((_BR,), lambda i: (i,)),
        ],
        out_shape=[
            jax.ShapeDtypeStruct((2, NP, HH), jnp.float32),
            jax.ShapeDtypeStruct((NP,), jnp.float32),
        ],
    )(x_p, cnt, Wn, bn, W0)


def _tc_mid_body(s_ref, g_ref, dinv_ref, b_ref, w_ref, o_ref):
    sv = jnp.concatenate([s_ref[0], s_ref[1]], axis=1)
    gv = jnp.concatenate([g_ref[0], g_ref[1]], axis=1)
    dinv = dinv_ref[...]
    hn = jnp.maximum((sv + gv) * dinv[:, None] + b_ref[...], 0.0)
    gn = (hn @ w_ref[...]) * dinv[:, None]
    for q in range(2):
        o_ref[q] = gn[:, q * HH:(q + 1) * HH]


def _tc_mid(s_pair, g_pair, dinv, b, Wnext):
    return pl.pallas_call(
        _tc_mid_body,
        grid=(_GRID,),
        in_specs=[
            pl.BlockSpec((2, _BR, HH), lambda i: (0, i, 0)),
            pl.BlockSpec((2, _BR, HH), lambda i: (0, i, 0)),
            pl.BlockSpec((_BR,), lambda i: (i,)),
            pl.BlockSpec((H,), lambda i: (0,)),
            pl.BlockSpec((H, H), lambda i: (0, 0)),
        ],
        out_specs=pl.BlockSpec((2, _BR, HH), lambda i: (0, i, 0)),
        out_shape=jax.ShapeDtypeStruct((2, NP, HH), jnp.float32),
    )(s_pair, g_pair, dinv, b, Wnext)


def _tc_last_body(s_ref, g_ref, dinv_ref, b_ref, o_ref):
    sv = jnp.concatenate([s_ref[0], s_ref[1]], axis=1)
    gv = jnp.concatenate([g_ref[0], g_ref[1]], axis=1)
    dinv = dinv_ref[...]
    hn = jnp.maximum((sv + gv) * dinv[:, None] + b_ref[...], 0.0)
    o_ref[0] = hn[:, :HH]
    o_ref[1] = hn[:, HH:]


def _tc_last(s_pair, g_pair, dinv, b):
    return pl.pallas_call(
        _tc_last_body,
        grid=(_GRID,),
        in_specs=[
            pl.BlockSpec((2, _BR, HH), lambda i: (0, i, 0)),
            pl.BlockSpec((2, _BR, HH), lambda i: (0, i, 0)),
            pl.BlockSpec((_BR,), lambda i: (i,)),
            pl.BlockSpec((H,), lambda i: (0,)),
        ],
        out_specs=pl.BlockSpec((2, _BR, HH), lambda i: (0, i, 0)),
        out_shape=jax.ShapeDtypeStruct((2, NP, HH), jnp.float32),
    )(s_pair, g_pair, dinv, b)


def _tc_head_body(sum_ref, max_ref, cnt_ref, w1_ref, b1_ref, w2_ref, b2_ref,
                  o_ref):
    sm = sum_ref[...]
    mx = max_ref[...]
    ct = cnt_ref[...]
    counts = jnp.sum(ct[0], axis=0)[:G, 0]
    s0 = jnp.sum(sm[0], axis=0)[:G]
    s1 = jnp.sum(sm[1], axis=0)[:G]
    m0 = jnp.max(mx[0], axis=0)[:G]
    m1 = jnp.max(mx[1], axis=0)[:G]
    inv = 1.0 / jnp.maximum(counts, 1.0)
    nz = counts > 0.0
    m0 = jnp.where(nz[:, None], m0, 0.0)
    m1 = jnp.where(nz[:, None], m1, 0.0)
    z = jnp.concatenate([s0 * inv[:, None], s1 * inv[:, None], m0, m1], axis=1)
    o = jnp.maximum(z @ w1_ref[...] + b1_ref[...], 0.0)
    o_ref[...] = o @ w2_ref[...] + b2_ref[...]


def _tc_head(sums, maxs, cnts, Wo1, bo1, Wo2, bo2):
    return pl.pallas_call(
        _tc_head_body,
        out_shape=jax.ShapeDtypeStruct((G, 1), jnp.float32),
    )(sums, maxs, cnts, Wo1, bo1, Wo2, bo2)


def kernel(x, edge_index, edge_attr, batch, Wn, bn, We, be, Wc, bc, Wo1, bo1,
           Wo2, bo2):
    L = Wc.shape[0]
    src = edge_index[0]
    dst = edge_index[1]
    pad = jnp.arange(EP - E, dtype=jnp.int32)
    src_p = jnp.concatenate([src, pad % 16]).reshape(16, 400, 128)
    dst_p = jnp.concatenate([dst, NP + (pad % 32)])
    dst_sc = dst_p.reshape(16, 400, 128)
    dst_deg = dst_p.reshape(2, 16, 200, 128)
    x_p = jnp.zeros((NP, 128), jnp.float32).at[:N].set(x)
    bpad = G + jnp.arange(NP - N, dtype=jnp.int32) % 16
    batch_p = jnp.concatenate([batch, bpad]).reshape(16, 25, 128)

    zeros16 = jnp.zeros((ROWS_PER_TILE, 16), jnp.float32)
    zeros32 = jnp.zeros((ROWS_PER_TILE, HH), jnp.float32)
    ones128 = jnp.ones((128, 16), jnp.float32)

    cnt = _sc_degrees(dst_deg, zeros16, ones128)
    g_pair, dinv = _tc_embed(x_p, cnt, Wn, bn, Wc[0])
    for i in range(L):
        s_pair = _sc_edge_scatter(g_pair, src_p, dst_sc, zeros32)
        if i < L - 1:
            g_pair = _tc_mid(s_pair, g_pair, dinv, bc[i], Wc[i + 1])
        else:
            h_pair = _tc_last(s_pair, g_pair, dinv, bc[i])
    sums, maxs, cnts = _sc_pool(h_pair, batch_p)
    out = _tc_head(sums, maxs, cnts, Wo1, bo1, Wo2, bo2)
    return out[:, 0]


# speedup vs baseline: 23.1398x; 1.0736x over previous
import functools

import jax
import jax.numpy as jnp
from jax import lax
from jax.experimental import pallas as pl
from jax.experimental.pallas import tpu as pltpu
from jax.experimental.pallas import tpu_sc as plsc

N = 50000
NP = 51200
NT = NP + 128
E = 800000
EP = 819200
G = 256
GP = 272
H = 64
HH = 32
ROWS_PER_TILE = NT // 16
NODE_ROWS_PER_TILE = NP // 16

_mesh = plsc.VectorSubcoreMesh(core_axis_name="c", subcore_axis_name="s")
_sc_params = pltpu.CompilerParams(use_tc_tiling_on_sc=False)


@functools.partial(
    pl.kernel,
    out_type=jax.ShapeDtypeStruct((2, NT, 16), jnp.float32),
    mesh=_mesh,
    compiler_params=_sc_params,
    scratch_types=[
        pltpu.VMEM((200, 128), jnp.int32),
        pltpu.VMEM((128, 16), jnp.float32),
        pltpu.VMEM_SHARED((NT, 16), jnp.float32),
        pltpu.SemaphoreType.DMA,
    ],
)
def _sc_degrees(dst_hbm, zeros_hbm, ones_hbm, out_hbm, idx_d, ones_v, acc, sem):
    c = lax.axis_index("c")
    s = lax.axis_index("s")
    r0 = s * ROWS_PER_TILE
    pltpu.sync_copy(zeros_hbm, acc.at[pl.ds(r0, ROWS_PER_TILE)])
    pltpu.sync_copy(ones_hbm, ones_v)
    pltpu.sync_copy(dst_hbm.at[c].at[s], idx_d)
    plsc.subcore_barrier()

    def step(i, _):
        for k in range(8):
            j = i * 8 + k
            pltpu.sync_copy(ones_v, acc.at[idx_d.at[j]], add=True)
        return 0

    lax.fori_loop(0, 25, step, 0)
    plsc.subcore_barrier()
    pltpu.sync_copy(acc.at[pl.ds(r0, ROWS_PER_TILE)],
                    out_hbm.at[c].at[pl.ds(r0, ROWS_PER_TILE)])


@functools.partial(
    pl.kernel,
    out_type=jax.ShapeDtypeStruct((2, NT, HH), jnp.float32),
    mesh=_mesh,
    compiler_params=_sc_params,
    scratch_types=[
        pltpu.VMEM((2, 20, 128), jnp.int32),
        pltpu.VMEM((2, 20, 128), jnp.int32),
        pltpu.VMEM((4, 128, HH), jnp.float32),
        pltpu.VMEM_SHARED((NT, HH), jnp.float32),
        pltpu.SemaphoreType.DMA,
        pltpu.SemaphoreType.DMA,
    ],
)
def _sc_edge_scatter(g_hbm, src_hbm, dst_hbm, zeros_hbm, out_hbm,
                     idx_s, idx_d, rows, acc, sem, sem2):
    c = lax.axis_index("c")
    s = lax.axis_index("s")
    r0 = s * ROWS_PER_TILE
    pltpu.sync_copy(zeros_hbm, acc.at[pl.ds(r0, ROWS_PER_TILE)])
    plsc.subcore_barrier()
    gq = g_hbm.at[c]
    dummy = gq.at[pl.ds(0, 128)]
    src_t = src_hbm.at[s]
    dst_t = dst_hbm.at[s]
    dummy_i = src_t.at[pl.ds(0, 20)]

    pltpu.sync_copy(src_t.at[pl.ds(0, 20)], idx_s.at[0])
    pltpu.sync_copy(dst_t.at[pl.ds(0, 20)], idx_d.at[0])
    for b in range(4):
        pltpu.async_copy(gq.at[idx_s.at[0].at[b]], rows.at[b], sem)

    for k in range(20):
        p = k % 2
        if k < 19:
            pltpu.async_copy(src_t.at[pl.ds((k + 1) * 20, 20)],
                             idx_s.at[1 - p], sem2)
            pltpu.async_copy(dst_t.at[pl.ds((k + 1) * 20, 20)],
                             idx_d.at[1 - p], sem2)
        isp = idx_s.at[p]
        idp = idx_d.at[p]

        def step(i, _):
            for b in range(4):
                t = i * 4 + b
                pltpu.make_async_copy(dummy, rows.at[b], sem).wait()
                pltpu.sync_copy(rows.at[b], acc.at[idp.at[t]], add=True)
                pltpu.async_copy(gq.at[isp.at[t + 4]], rows.at[b], sem)
            return 0

        lax.fori_loop(0, 4, step, 0)
        if k < 19:
            pltpu.make_async_copy(dummy_i, idx_s.at[1 - p], sem2).wait()
            pltpu.make_async_copy(dummy_i, idx_d.at[1 - p], sem2).wait()
            nsp = idx_s.at[1 - p]
            for b in range(4):
                t = 16 + b
                pltpu.make_async_copy(dummy, rows.at[b], sem).wait()
                pltpu.sync_copy(rows.at[b], acc.at[idp.at[t]], add=True)
                pltpu.async_copy(gq.at[nsp.at[b]], rows.at[b], sem)
        else:
            for b in range(4):
                t = 16 + b
                pltpu.make_async_copy(dummy, rows.at[b], sem).wait()
                pltpu.sync_copy(rows.at[b], acc.at[idp.at[t]], add=True)
    plsc.subcore_barrier()
    pltpu.sync_copy(acc.at[pl.ds(r0, ROWS_PER_TILE)],
                    out_hbm.at[c].at[pl.ds(r0, ROWS_PER_TILE)])


@functools.partial(
    pl.kernel,
    out_type=[
        jax.ShapeDtypeStruct((2, 16, GP, HH), jnp.float32),
        jax.ShapeDtypeStruct((2, 16, GP, HH), jnp.float32),
        jax.ShapeDtypeStruct((2, 16, GP, 16), jnp.float32),
    ],
    mesh=_mesh,
    compiler_params=_sc_params,
    scratch_types=[
        pltpu.VMEM((25, 128), jnp.int32),
        pltpu.VMEM((128, HH), jnp.float32),
        pltpu.VMEM((GP, HH), jnp.float32),
        pltpu.VMEM((GP, HH), jnp.float32),
        pltpu.VMEM((GP, 16), jnp.float32),
        pltpu.SemaphoreType.DMA,
    ],
)
def _sc_pool(h_hbm, batch_hbm, sum_hbm, max_hbm, cnt_hbm,
             bidx, win, sacc, macc, cacc, sem):
    c = lax.axis_index("c")
    s = lax.axis_index("s")
    pltpu.sync_copy(batch_hbm.at[s], bidx)
    zero16 = jnp.zeros((16,), jnp.float32)
    neg16 = jnp.full((16,), -1e30, jnp.float32)

    def init(i, _):
        sacc[i, pl.ds(0, 16)] = zero16
        sacc[i, pl.ds(16, 16)] = zero16
        macc[i, pl.ds(0, 16)] = neg16
        macc[i, pl.ds(16, 16)] = neg16
        cacc[i, pl.ds(0, 16)] = zero16
        return 0

    lax.fori_loop(0, GP, init, 0)
    hc = h_hbm.at[c]
    base = s * NODE_ROWS_PER_TILE

    def window(w, _):
        pltpu.sync_copy(hc.at[pl.ds(base + w * 128, 128)], win)

        def rowgrp(rr, _):
            bvec = bidx[w, pl.ds(rr * 16, 16)]
            for k in range(16):
                b = bvec[k]
                r = rr * 16 + k
                v0 = win[r, pl.ds(0, 16)]
                v1 = win[r, pl.ds(16, 16)]
                sacc[b, pl.ds(0, 16)] = sacc[b, pl.ds(0, 16)] + v0
                sacc[b, pl.ds(16, 16)] = sacc[b, pl.ds(16, 16)] + v1
                macc[b, pl.ds(0, 16)] = jnp.maximum(macc[b, pl.ds(0, 16)], v0)
                macc[b, pl.ds(16, 16)] = jnp.maximum(macc[b, pl.ds(16, 16)], v1)
                cacc[b, pl.ds(0, 16)] = cacc[b, pl.ds(0, 16)] + 1.0
            return 0

        lax.fori_loop(0, 8, rowgrp, 0)
        return 0

    lax.fori_loop(0, 25, window, 0)
    pltpu.sync_copy(sacc, sum_hbm.at[c].at[s])
    pltpu.sync_copy(macc, max_hbm.at[c].at[s])
    pltpu.sync_copy(cacc, cnt_hbm.at[c].at[s])


_BR = 1024
_GRID = 49


def _tc_embed_body(x_ref, cnt_ref, wn_ref, bn_ref, w0_ref, g_ref, dinv_ref):
    cnt = cnt_ref[...]
    deg = 1.0 + cnt[0, :, 0] + cnt[1, :, 0]
    dinv = lax.rsqrt(deg)
    h = x_ref[...] @ wn_ref[...] + bn_ref[...]
    g = (h @ w0_ref[...]) * dinv[:, None]
    for q in range(2):
        g_ref[q] = g[:, q * HH:(q + 1) * HH]
    dinv_ref[...] = dinv


def _tc_embed(x, cnt, Wn, bn, W0):
    return pl.pallas_call(
        _tc_embed_body,
        grid=(_GRID,),
        in_specs=[
            pl.BlockSpec((_BR, 128), lambda i: (i, 0)),
            pl.BlockSpec((2, _BR, 16), lambda i: (0, i, 0)),
            pl.BlockSpec((128, H), lambda i: (0, 0)),
            pl.BlockSpec((H,), lambda i: (0,)),
            pl.BlockSpec((H, H), lambda i: (0, 0)),
        ],
        out_specs=[
            pl.BlockSpec((2, _BR, HH), lambda i: (0, i, 0)),
            pl.BlockSpec((_BR,), lambda i: (i,)),
        ],
        out_shape=[
            jax.ShapeDtypeStruct((2, NP, HH), jnp.float32),
            jax.ShapeDtypeStruct((NP,), jnp.float32),
        ],
    )(x, cnt, Wn, bn, W0)


def _tc_mid_body(s_ref, g_ref, dinv_ref, b_ref, w_ref, o_ref):
    sv = jnp.concatenate([s_ref[0], s_ref[1]], axis=1)
    gv = jnp.concatenate([g_ref[0], g_ref[1]], axis=1)
    dinv = dinv_ref[...]
    hn = jnp.maximum((sv + gv) * dinv[:, None] + b_ref[...], 0.0)
    gn = (hn @ w_ref[...]) * dinv[:, None]
    for q in range(2):
        o_ref[q] = gn[:, q * HH:(q + 1) * HH]


def _tc_mid(s_pair, g_pair, dinv, b, Wnext):
    return pl.pallas_call(
        _tc_mid_body,
        grid=(_GRID,),
        in_specs=[
            pl.BlockSpec((2, _BR, HH), lambda i: (0, i, 0)),
            pl.BlockSpec((2, _BR, HH), lambda i: (0, i, 0)),
            pl.BlockSpec((_BR,), lambda i: (i,)),
            pl.BlockSpec((H,), lambda i: (0,)),
            pl.BlockSpec((H, H), lambda i: (0, 0)),
        ],
        out_specs=pl.BlockSpec((2, _BR, HH), lambda i: (0, i, 0)),
        out_shape=jax.ShapeDtypeStruct((2, NP, HH), jnp.float32),
    )(s_pair, g_pair, dinv, b, Wnext)


def _tc_last_body(s_ref, g_ref, dinv_ref, b_ref, o_ref):
    sv = jnp.concatenate([s_ref[0], s_ref[1]], axis=1)
    gv = jnp.concatenate([g_ref[0], g_ref[1]], axis=1)
    dinv = dinv_ref[...]
    hn = jnp.maximum((sv + gv) * dinv[:, None] + b_ref[...], 0.0)
    o_ref[0] = hn[:, :HH]
    o_ref[1] = hn[:, HH:]


def _tc_last(s_pair, g_pair, dinv, b):
    return pl.pallas_call(
        _tc_last_body,
        grid=(_GRID,),
        in_specs=[
            pl.BlockSpec((2, _BR, HH), lambda i: (0, i, 0)),
            pl.BlockSpec((2, _BR, HH), lambda i: (0, i, 0)),
            pl.BlockSpec((_BR,), lambda i: (i,)),
            pl.BlockSpec((H,), lambda i: (0,)),
        ],
        out_specs=pl.BlockSpec((2, _BR, HH), lambda i: (0, i, 0)),
        out_shape=jax.ShapeDtypeStruct((2, NP, HH), jnp.float32),
    )(s_pair, g_pair, dinv, b)


def _tc_head_body(sum_ref, max_ref, cnt_ref, w1_ref, b1_ref, w2_ref, b2_ref,
                  o_ref):
    sm = sum_ref[...]
    mx = max_ref[...]
    ct = cnt_ref[...]
    counts = jnp.sum(ct[0], axis=0)[:G, 0]
    s0 = jnp.sum(sm[0], axis=0)[:G]
    s1 = jnp.sum(sm[1], axis=0)[:G]
    m0 = jnp.max(mx[0], axis=0)[:G]
    m1 = jnp.max(mx[1], axis=0)[:G]
    inv = 1.0 / jnp.maximum(counts, 1.0)
    nz = counts > 0.0
    m0 = jnp.where(nz[:, None], m0, 0.0)
    m1 = jnp.where(nz[:, None], m1, 0.0)
    z = jnp.concatenate([s0 * inv[:, None], s1 * inv[:, None], m0, m1], axis=1)
    o = jnp.maximum(z @ w1_ref[...] + b1_ref[...], 0.0)
    o_ref[...] = o @ w2_ref[...] + b2_ref[...]


def _tc_head(sums, maxs, cnts, Wo1, bo1, Wo2, bo2):
    return pl.pallas_call(
        _tc_head_body,
        out_shape=jax.ShapeDtypeStruct((G, 1), jnp.float32),
    )(sums, maxs, cnts, Wo1, bo1, Wo2, bo2)


def kernel(x, edge_index, edge_attr, batch, Wn, bn, We, be, Wc, bc, Wo1, bo1,
           Wo2, bo2):
    L = Wc.shape[0]
    src = edge_index[0]
    dst = edge_index[1]
    pad = jnp.arange(EP - E, dtype=jnp.int32)
    src_p = jnp.concatenate([src, pad % 16]).reshape(16, 400, 128)
    dst_p = jnp.concatenate([dst, NP + (pad % 32)])
    dst_sc = dst_p.reshape(16, 400, 128)
    dst_deg = dst_p.reshape(2, 16, 200, 128)
    bpad = G + jnp.arange(NP - N, dtype=jnp.int32) % 16
    batch_p = jnp.concatenate([batch, bpad]).reshape(16, 25, 128)

    zeros16 = jnp.zeros((ROWS_PER_TILE, 16), jnp.float32)
    zeros32 = jnp.zeros((ROWS_PER_TILE, HH), jnp.float32)
    ones128 = jnp.ones((128, 16), jnp.float32)

    cnt = _sc_degrees(dst_deg, zeros16, ones128)
    g_pair, dinv = _tc_embed(x, cnt, Wn, bn, Wc[0])
    for i in range(L):
        s_pair = _sc_edge_scatter(g_pair, src_p, dst_sc, zeros32)
        if i < L - 1:
            g_pair = _tc_mid(s_pair, g_pair, dinv, bc[i], Wc[i + 1])
        else:
            h_pair = _tc_last(s_pair, g_pair, dinv, bc[i])
    sums, maxs, cnts = _sc_pool(h_pair, batch_p)
    out = _tc_head(sums, maxs, cnts, Wo1, bo1, Wo2, bo2)
    return out[:, 0]


# speedup vs baseline: 23.9536x; 1.0352x over previous
import functools

import jax
import jax.numpy as jnp
from jax import lax
from jax.experimental import pallas as pl
from jax.experimental.pallas import tpu as pltpu
from jax.experimental.pallas import tpu_sc as plsc

N = 50000
NP = 51200
NT = NP + 128
E = 800000
EP = 819200
G = 256
GP = 272
H = 64
HH = 32
ROWS_PER_TILE = NT // 16
NODE_ROWS_PER_TILE = NP // 16

_mesh = plsc.VectorSubcoreMesh(core_axis_name="c", subcore_axis_name="s")
_sc_params = pltpu.CompilerParams(use_tc_tiling_on_sc=False)


@functools.partial(
    pl.kernel,
    out_type=jax.ShapeDtypeStruct((2, NT, 16), jnp.float32),
    mesh=_mesh,
    compiler_params=_sc_params,
    scratch_types=[
        pltpu.VMEM((200, 128), jnp.int32),
        pltpu.VMEM((128, 16), jnp.float32),
        pltpu.VMEM_SHARED((NT, 16), jnp.float32),
        pltpu.SemaphoreType.DMA,
    ],
)
def _sc_degrees(dst_hbm, zeros_hbm, ones_hbm, out_hbm, idx_d, ones_v, acc, sem):
    c = lax.axis_index("c")
    s = lax.axis_index("s")
    r0 = s * ROWS_PER_TILE
    pltpu.sync_copy(zeros_hbm, acc.at[pl.ds(r0, ROWS_PER_TILE)])
    pltpu.sync_copy(ones_hbm, ones_v)
    pltpu.sync_copy(dst_hbm.at[c].at[s], idx_d)
    plsc.subcore_barrier()

    def step(i, _):
        for k in range(8):
            j = i * 8 + k
            pltpu.sync_copy(ones_v, acc.at[idx_d.at[j]], add=True)
        return 0

    lax.fori_loop(0, 25, step, 0)
    plsc.subcore_barrier()
    pltpu.sync_copy(acc.at[pl.ds(r0, ROWS_PER_TILE)],
                    out_hbm.at[c].at[pl.ds(r0, ROWS_PER_TILE)])


@functools.partial(
    pl.kernel,
    out_type=jax.ShapeDtypeStruct((2, NT, HH), jnp.float32),
    mesh=_mesh,
    compiler_params=_sc_params,
    scratch_types=[
        pltpu.VMEM((2, 10, 128), jnp.int32),
        pltpu.VMEM((2, 10, 128), jnp.int32),
        pltpu.VMEM((5, 128, HH), jnp.float32),
        pltpu.VMEM_SHARED((NT, HH), jnp.float32),
        pltpu.SemaphoreType.DMA,
        pltpu.SemaphoreType.DMA,
    ],
)
def _sc_edge_scatter(g_hbm, src_hbm, dst_hbm, zeros_hbm, out_hbm,
                     idx_s, idx_d, rows, acc, sem, sem2):
    c = lax.axis_index("c")
    s = lax.axis_index("s")
    r0 = s * ROWS_PER_TILE
    pltpu.sync_copy(zeros_hbm, acc.at[pl.ds(r0, ROWS_PER_TILE)])
    plsc.subcore_barrier()
    gq = g_hbm.at[c]
    dummy = gq.at[pl.ds(0, 128)]
    src_t = src_hbm.at[s]
    dst_t = dst_hbm.at[s]
    dummy_i = src_t.at[pl.ds(0, 10)]

    pltpu.sync_copy(src_t.at[pl.ds(0, 10)], idx_s.at[0])
    pltpu.sync_copy(dst_t.at[pl.ds(0, 10)], idx_d.at[0])
    for b in range(5):
        pltpu.async_copy(gq.at[idx_s.at[0].at[b]], rows.at[b], sem)

    for k in range(40):
        p = k % 2
        if k < 39:
            pltpu.async_copy(src_t.at[pl.ds((k + 1) * 10, 10)],
                             idx_s.at[1 - p], sem2)
            pltpu.async_copy(dst_t.at[pl.ds((k + 1) * 10, 10)],
                             idx_d.at[1 - p], sem2)
        isp = idx_s.at[p]
        idp = idx_d.at[p]

        for t in range(5):
            b = t
            pltpu.make_async_copy(dummy, rows.at[b], sem).wait()
            pltpu.sync_copy(rows.at[b], acc.at[idp.at[t]], add=True)
            pltpu.async_copy(gq.at[isp.at[t + 5]], rows.at[b], sem)
        if k < 39:
            pltpu.make_async_copy(dummy_i, idx_s.at[1 - p], sem2).wait()
            pltpu.make_async_copy(dummy_i, idx_d.at[1 - p], sem2).wait()
            nsp = idx_s.at[1 - p]
            for t in range(5, 10):
                b = t - 5
                pltpu.make_async_copy(dummy, rows.at[b], sem).wait()
                pltpu.sync_copy(rows.at[b], acc.at[idp.at[t]], add=True)
                pltpu.async_copy(gq.at[nsp.at[t - 5]], rows.at[b], sem)
        else:
            for t in range(5, 10):
                b = t - 5
                pltpu.make_async_copy(dummy, rows.at[b], sem).wait()
                pltpu.sync_copy(rows.at[b], acc.at[idp.at[t]], add=True)
    plsc.subcore_barrier()
    pltpu.sync_copy(acc.at[pl.ds(r0, ROWS_PER_TILE)],
                    out_hbm.at[c].at[pl.ds(r0, ROWS_PER_TILE)])


@functools.partial(
    pl.kernel,
    out_type=[
        jax.ShapeDtypeStruct((2, 16, GP, HH), jnp.float32),
        jax.ShapeDtypeStruct((2, 16, GP, HH), jnp.float32),
        jax.ShapeDtypeStruct((2, 16, GP, 16), jnp.float32),
    ],
    mesh=_mesh,
    compiler_params=_sc_params,
    scratch_types=[
        pltpu.VMEM((25, 128), jnp.int32),
        pltpu.VMEM((128, HH), jnp.float32),
        pltpu.VMEM((GP, HH), jnp.float32),
        pltpu.VMEM((GP, HH), jnp.float32),
        pltpu.VMEM((GP, 16), jnp.float32),
        pltpu.SemaphoreType.DMA,
    ],
)
def _sc_pool(h_hbm, batch_hbm, sum_hbm, max_hbm, cnt_hbm,
             bidx, win, sacc, macc, cacc, sem):
    c = lax.axis_index("c")
    s = lax.axis_index("s")
    pltpu.sync_copy(batch_hbm.at[s], bidx)
    zero16 = jnp.zeros((16,), jnp.float32)
    neg16 = jnp.full((16,), -1e30, jnp.float32)

    def init(i, _):
        sacc[i, pl.ds(0, 16)] = zero16
        sacc[i, pl.ds(16, 16)] = zero16
        macc[i, pl.ds(0, 16)] = neg16
        macc[i, pl.ds(16, 16)] = neg16
        cacc[i, pl.ds(0, 16)] = zero16
        return 0

    lax.fori_loop(0, GP, init, 0)
    hc = h_hbm.at[c]
    base = s * NODE_ROWS_PER_TILE

    def window(w, _):
        pltpu.sync_copy(hc.at[pl.ds(base + w * 128, 128)], win)

        def rowgrp(rr, _):
            bvec = bidx[w, pl.ds(rr * 16, 16)]
            for k in range(16):
                b = bvec[k]
                r = rr * 16 + k
                v0 = win[r, pl.ds(0, 16)]
                v1 = win[r, pl.ds(16, 16)]
                sacc[b, pl.ds(0, 16)] = sacc[b, pl.ds(0, 16)] + v0
                sacc[b, pl.ds(16, 16)] = sacc[b, pl.ds(16, 16)] + v1
                macc[b, pl.ds(0, 16)] = jnp.maximum(macc[b, pl.ds(0, 16)], v0)
                macc[b, pl.ds(16, 16)] = jnp.maximum(macc[b, pl.ds(16, 16)], v1)
                cacc[b, pl.ds(0, 16)] = cacc[b, pl.ds(0, 16)] + 1.0
            return 0

        lax.fori_loop(0, 8, rowgrp, 0)
        return 0

    lax.fori_loop(0, 25, window, 0)
    pltpu.sync_copy(sacc, sum_hbm.at[c].at[s])
    pltpu.sync_copy(macc, max_hbm.at[c].at[s])
    pltpu.sync_copy(cacc, cnt_hbm.at[c].at[s])


_BR = 1024
_GRID = 49


def _tc_embed_body(x_ref, cnt_ref, wn_ref, bn_ref, w0_ref, g_ref, dinv_ref):
    cnt = cnt_ref[...]
    deg = 1.0 + cnt[0, :, 0] + cnt[1, :, 0]
    dinv = lax.rsqrt(deg)
    h = x_ref[...] @ wn_ref[...] + bn_ref[...]
    g = (h @ w0_ref[...]) * dinv[:, None]
    for q in range(2):
        g_ref[q] = g[:, q * HH:(q + 1) * HH]
    dinv_ref[...] = dinv


def _tc_embed(x, cnt, Wn, bn, W0):
    return pl.pallas_call(
        _tc_embed_body,
        grid=(_GRID,),
        in_specs=[
            pl.BlockSpec((_BR, 128), lambda i: (i, 0)),
            pl.BlockSpec((2, _BR, 16), lambda i: (0, i, 0)),
            pl.BlockSpec((128, H), lambda i: (0, 0)),
            pl.BlockSpec((H,), lambda i: (0,)),
            pl.BlockSpec((H, H), lambda i: (0, 0)),
        ],
        out_specs=[
            pl.BlockSpec((2, _BR, HH), lambda i: (0, i, 0)),
            pl.BlockSpec((_BR,), lambda i: (i,)),
        ],
        out_shape=[
            jax.ShapeDtypeStruct((2, NP, HH), jnp.float32),
            jax.ShapeDtypeStruct((NP,), jnp.float32),
        ],
    )(x, cnt, Wn, bn, W0)


def _tc_mid_body(s_ref, g_ref, dinv_ref, b_ref, w_ref, o_ref):
    sv = jnp.concatenate([s_ref[0], s_ref[1]], axis=1)
    gv = jnp.concatenate([g_ref[0], g_ref[1]], axis=1)
    dinv = dinv_ref[...]
    hn = jnp.maximum((sv + gv) * dinv[:, None] + b_ref[...], 0.0)
    gn = (hn @ w_ref[...]) * dinv[:, None]
    for q in range(2):
        o_ref[q] = gn[:, q * HH:(q + 1) * HH]


def _tc_mid(s_pair, g_pair, dinv, b, Wnext):
    return pl.pallas_call(
        _tc_mid_body,
        grid=(_GRID,),
        in_specs=[
            pl.BlockSpec((2, _BR, HH), lambda i: (0, i, 0)),
            pl.BlockSpec((2, _BR, HH), lambda i: (0, i, 0)),
            pl.BlockSpec((_BR,), lambda i: (i,)),
            pl.BlockSpec((H,), lambda i: (0,)),
            pl.BlockSpec((H, H), lambda i: (0, 0)),
        ],
        out_specs=pl.BlockSpec((2, _BR, HH), lambda i: (0, i, 0)),
        out_shape=jax.ShapeDtypeStruct((2, NP, HH), jnp.float32),
    )(s_pair, g_pair, dinv, b, Wnext)


def _tc_last_body(s_ref, g_ref, dinv_ref, b_ref, o_ref):
    sv = jnp.concatenate([s_ref[0], s_ref[1]], axis=1)
    gv = jnp.concatenate([g_ref[0], g_ref[1]], axis=1)
    dinv = dinv_ref[...]
    hn = jnp.maximum((sv + gv) * dinv[:, None] + b_ref[...], 0.0)
    o_ref[0] = hn[:, :HH]
    o_ref[1] = hn[:, HH:]


def _tc_last(s_pair, g_pair, dinv, b):
    return pl.pallas_call(
        _tc_last_body,
        grid=(_GRID,),
        in_specs=[
            pl.BlockSpec((2, _BR, HH), lambda i: (0, i, 0)),
            pl.BlockSpec((2, _BR, HH), lambda i: (0, i, 0)),
            pl.BlockSpec((_BR,), lambda i: (i,)),
            pl.BlockSpec((H,), lambda i: (0,)),
        ],
        out_specs=pl.BlockSpec((2, _BR, HH), lambda i: (0, i, 0)),
        out_shape=jax.ShapeDtypeStruct((2, NP, HH), jnp.float32),
    )(s_pair, g_pair, dinv, b)


def _tc_head_body(sum_ref, max_ref, cnt_ref, w1_ref, b1_ref, w2_ref, b2_ref,
                  o_ref):
    sm = sum_ref[...]
    mx = max_ref[...]
    ct = cnt_ref[...]
    counts = jnp.sum(ct[0], axis=0)[:G, 0]
    s0 = jnp.sum(sm[0], axis=0)[:G]
    s1 = jnp.sum(sm[1], axis=0)[:G]
    m0 = jnp.max(mx[0], axis=0)[:G]
    m1 = jnp.max(mx[1], axis=0)[:G]
    inv = 1.0 / jnp.maximum(counts, 1.0)
    nz = counts > 0.0
    m0 = jnp.where(nz[:, None], m0, 0.0)
    m1 = jnp.where(nz[:, None], m1, 0.0)
    z = jnp.concatenate([s0 * inv[:, None], s1 * inv[:, None], m0, m1], axis=1)
    o = jnp.maximum(z @ w1_ref[...] + b1_ref[...], 0.0)
    o_ref[...] = o @ w2_ref[...] + b2_ref[...]


def _tc_head(sums, maxs, cnts, Wo1, bo1, Wo2, bo2):
    return pl.pallas_call(
        _tc_head_body,
        out_shape=jax.ShapeDtypeStruct((G, 1), jnp.float32),
    )(sums, maxs, cnts, Wo1, bo1, Wo2, bo2)


def kernel(x, edge_index, edge_attr, batch, Wn, bn, We, be, Wc, bc, Wo1, bo1,
           Wo2, bo2):
    L = Wc.shape[0]
    src = edge_index[0]
    dst = edge_index[1]
    pad = jnp.arange(EP - E, dtype=jnp.int32)
    src_p = jnp.concatenate([src, pad % 16]).reshape(16, 400, 128)
    dst_p = jnp.concatenate([dst, NP + (pad % 32)])
    dst_sc = dst_p.reshape(16, 400, 128)
    dst_deg = dst_p.reshape(2, 16, 200, 128)
    bpad = G + jnp.arange(NP - N, dtype=jnp.int32) % 16
    batch_p = jnp.concatenate([batch, bpad]).reshape(16, 25, 128)

    zeros16 = jnp.zeros((ROWS_PER_TILE, 16), jnp.float32)
    zeros32 = jnp.zeros((ROWS_PER_TILE, HH), jnp.float32)
    ones128 = jnp.ones((128, 16), jnp.float32)

    cnt = _sc_degrees(dst_deg, zeros16, ones128)
    g_pair, dinv = _tc_embed(x, cnt, Wn, bn, Wc[0])
    for i in range(L):
        s_pair = _sc_edge_scatter(g_pair, src_p, dst_sc, zeros32)
        if i < L - 1:
            g_pair = _tc_mid(s_pair, g_pair, dinv, bc[i], Wc[i + 1])
        else:
            h_pair = _tc_last(s_pair, g_pair, dinv, bc[i])
    sums, maxs, cnts = _sc_pool(h_pair, batch_p)
    out = _tc_head(sums, maxs, cnts, Wo1, bo1, Wo2, bo2)
    return out[:, 0]
